# baseline (speedup 1.0000x reference)
import jax
import jax.numpy as jnp
from jax.experimental import pallas as pl

_B, _S, _E = 16384, 200, 64
_R = 16


def _body(ts_ref, emb_ref, out_ref):
    ts = ts_ref[...]
    a = ts[:, :, None]
    b = ts[:, None, :]
    k2 = jax.lax.broadcasted_iota(jnp.int32, (_S, _S), 1)
    j2 = jax.lax.broadcasted_iota(jnp.int32, (_S, _S), 0)
    tri = (k2 < j2)[None, :, :]
    cmp = (b < a) | ((b == a) & tri)
    rank = jnp.sum(cmp.astype(jnp.int32), axis=2)
    i2 = jax.lax.broadcasted_iota(jnp.int32, (_S, _S), 0)[None, :, :]
    p = (i2 == rank[:, None, :]).astype(jnp.bfloat16)
    out = jnp.dot(p.reshape(_R * _S, _S), emb_ref[...].astype(jnp.bfloat16),
                  preferred_element_type=jnp.float32)
    out_ref[...] = out.reshape(_R, _S, _E)


def kernel(ts, pos_embeddings):
    return pl.pallas_call(
        _body,
        grid=(_B // _R,),
        in_specs=[
            pl.BlockSpec((_R, _S), lambda i: (i, 0)),
            pl.BlockSpec((_S, _E), lambda i: (0, 0)),
        ],
        out_specs=pl.BlockSpec((_R, _S, _E), lambda i: (i, 0, 0)),
        out_shape=jax.ShapeDtypeStruct((_B, _S, _E), jnp.float32),
    )(ts, pos_embeddings)


# speedup vs baseline: 1.0445x; 1.0445x over previous
import jax
import jax.numpy as jnp
from jax.experimental import pallas as pl

_B, _S, _E = 16384, 200, 64
_R = 16


def _body(ts_ref, emb_ref, out_ref):
    ts = ts_ref[...]
    a = ts[:, :, None]
    b = ts[:, None, :]
    k2 = jax.lax.broadcasted_iota(jnp.int32, (_S, _S), 1)
    j2 = jax.lax.broadcasted_iota(jnp.int32, (_S, _S), 0)
    tri = (k2 < j2)[None, :, :]
    cmp = (b < a) | ((b <= a) & tri)
    rank = jnp.sum(cmp.astype(jnp.float32), axis=2)
    i2 = j2.astype(jnp.float32)[None, :, :]
    p = (i2 == rank[:, None, :]).astype(jnp.bfloat16)
    out = jnp.dot(p.reshape(_R * _S, _S), emb_ref[...].astype(jnp.bfloat16),
                  preferred_element_type=jnp.float32)
    out_ref[...] = out.reshape(_R, _S, _E)


def kernel(ts, pos_embeddings):
    return pl.pallas_call(
        _body,
        grid=(_B // _R,),
        in_specs=[
            pl.BlockSpec((_R, _S), lambda i: (i, 0)),
            pl.BlockSpec((_S, _E), lambda i: (0, 0)),
        ],
        out_specs=pl.BlockSpec((_R, _S, _E), lambda i: (i, 0, 0)),
        out_shape=jax.ShapeDtypeStruct((_B, _S, _E), jnp.float32),
    )(ts, pos_embeddings)


# speedup vs baseline: 1.2163x; 1.1646x over previous
import jax
import jax.numpy as jnp
from jax.experimental import pallas as pl

_B, _S, _E = 16384, 200, 64
_R = 16


def _body(ts_ref, emb_ref, out_ref):
    ts = ts_ref[...]
    a = ts[:, :, None]
    b = ts[:, None, :]
    k2 = jax.lax.broadcasted_iota(jnp.int32, (_S, _S), 1)
    j2 = jax.lax.broadcasted_iota(jnp.int32, (_S, _S), 0)
    tri = (k2 < j2)[None, :, :]
    cmp = (b < a) | ((b <= a) & tri)
    rank = jnp.sum(cmp.astype(jnp.float32), axis=2)
    i2 = j2.astype(jnp.float32)[None, :, :]
    p = (i2 == rank[:, None, :]).astype(jnp.bfloat16)
    out = jnp.dot(p.reshape(_R * _S, _S), emb_ref[...].astype(jnp.bfloat16),
                  preferred_element_type=jnp.float32)
    out_ref[...] = out


def kernel(ts, pos_embeddings):
    out = pl.pallas_call(
        _body,
        grid=(_B // _R,),
        in_specs=[
            pl.BlockSpec((_R, _S), lambda i: (i, 0)),
            pl.BlockSpec((_S, _E), lambda i: (0, 0)),
        ],
        out_specs=pl.BlockSpec((_R * _S, _E), lambda i: (i, 0)),
        out_shape=jax.ShapeDtypeStruct((_B * _S, _E), jnp.float32),
    )(ts, pos_embeddings)
    return out.reshape(_B, _S, _E)


# speedup vs baseline: 1.2868x; 1.0579x over previous
import jax
import jax.numpy as jnp
from jax.experimental import pallas as pl

_B, _S, _E = 16384, 200, 64
_R = 16


def _body(ts_ref, emb_ref, out_ref):
    ts = ts_ref[...]
    ib = jax.lax.bitcast_convert_type(ts, jnp.int32)
    ikey = ib ^ ((ib >> 31) & jnp.int32(0x7FFFFFFF))
    ikey = jnp.where(ts == 0.0, 0, ikey)
    a = ikey[:, :, None]
    b = ikey[:, None, :]
    k2 = jax.lax.broadcasted_iota(jnp.int32, (_S, _S), 1)
    j2 = jax.lax.broadcasted_iota(jnp.int32, (_S, _S), 0)
    tri01 = (k2 < j2).astype(jnp.int32)[None, :, :]
    cmp = b < (a + tri01)
    rank = jnp.sum(cmp.astype(jnp.float32), axis=2)
    i2 = j2.astype(jnp.float32)[None, :, :]
    p = (i2 == rank[:, None, :]).astype(jnp.bfloat16)
    out = jnp.dot(p.reshape(_R * _S, _S), emb_ref[...].astype(jnp.bfloat16),
                  preferred_element_type=jnp.float32)
    out_ref[...] = out


def kernel(ts, pos_embeddings):
    out = pl.pallas_call(
        _body,
        grid=(_B // _R,),
        in_specs=[
            pl.BlockSpec((_R, _S), lambda i: (i, 0)),
            pl.BlockSpec((_S, _E), lambda i: (0, 0)),
        ],
        out_specs=pl.BlockSpec((_R * _S, _E), lambda i: (i, 0)),
        out_shape=jax.ShapeDtypeStruct((_B * _S, _E), jnp.float32),
    )(ts, pos_embeddings)
    return out.reshape(_B, _S, _E)


# speedup vs baseline: 1.5119x; 1.1749x over previous
import jax
import jax.numpy as jnp
from jax.experimental import pallas as pl

_B, _S, _E = 16384, 200, 64
_R = 32


def _body(ts_ref, emb_ref, out_ref):
    ts = ts_ref[...]
    ib = jax.lax.bitcast_convert_type(ts, jnp.int32)
    ikey = ib ^ ((ib >> 31) & jnp.int32(0x7FFFFFFF))
    ikey = jnp.where(ts == 0.0, 0, ikey)
    a = ikey[:, :, None]
    b = ikey[:, None, :]
    k2 = jax.lax.broadcasted_iota(jnp.int32, (_S, _S), 1)
    j2 = jax.lax.broadcasted_iota(jnp.int32, (_S, _S), 0)
    tri01 = (k2 < j2).astype(jnp.int32)[None, :, :]
    cmp = b < (a + tri01)
    rank = jnp.sum(cmp.astype(jnp.float32), axis=2)
    i2 = j2.astype(jnp.float32)[None, :, :]
    p = (i2 == rank[:, None, :]).astype(jnp.bfloat16)
    out = jnp.dot(p.reshape(_R * _S, _S), emb_ref[...].astype(jnp.bfloat16),
                  preferred_element_type=jnp.float32)
    out_ref[...] = out


def kernel(ts, pos_embeddings):
    out = pl.pallas_call(
        _body,
        grid=(_B // _R,),
        in_specs=[
            pl.BlockSpec((_R, _S), lambda i: (i, 0)),
            pl.BlockSpec((_S, _E), lambda i: (0, 0)),
        ],
        out_specs=pl.BlockSpec((_R * _S, _E), lambda i: (i, 0)),
        out_shape=jax.ShapeDtypeStruct((_B * _S, _E), jnp.float32),
    )(ts, pos_embeddings)
    return out.reshape(_B, _S, _E)


# speedup vs baseline: 1.6573x; 1.0962x over previous
import jax
import jax.numpy as jnp
from jax.experimental import pallas as pl

_B, _S, _E = 16384, 200, 64
_R = 64


def _body(ts_ref, emb_ref, out_ref):
    ts = ts_ref[...]
    ib = jax.lax.bitcast_convert_type(ts, jnp.int32)
    ikey = ib ^ ((ib >> 31) & jnp.int32(0x7FFFFFFF))
    ikey = jnp.where(ts == 0.0, 0, ikey)
    a = ikey[:, :, None]
    b = ikey[:, None, :]
    k2 = jax.lax.broadcasted_iota(jnp.int32, (_S, _S), 1)
    j2 = jax.lax.broadcasted_iota(jnp.int32, (_S, _S), 0)
    tri01 = (k2 < j2).astype(jnp.int32)[None, :, :]
    cmp = b < (a + tri01)
    rank = jnp.sum(cmp.astype(jnp.float32), axis=2)
    i2 = j2.astype(jnp.float32)[None, :, :]
    p = (i2 == rank[:, None, :]).astype(jnp.bfloat16)
    out = jnp.dot(p.reshape(_R * _S, _S), emb_ref[...].astype(jnp.bfloat16),
                  preferred_element_type=jnp.float32)
    out_ref[...] = out


def kernel(ts, pos_embeddings):
    out = pl.pallas_call(
        _body,
        grid=(_B // _R,),
        in_specs=[
            pl.BlockSpec((_R, _S), lambda i: (i, 0)),
            pl.BlockSpec((_S, _E), lambda i: (0, 0)),
        ],
        out_specs=pl.BlockSpec((_R * _S, _E), lambda i: (i, 0)),
        out_shape=jax.ShapeDtypeStruct((_B * _S, _E), jnp.float32),
    )(ts, pos_embeddings)
    return out.reshape(_B, _S, _E)


# speedup vs baseline: 1.7326x; 1.0454x over previous
import jax
import jax.numpy as jnp
from jax.experimental import pallas as pl

_B, _S, _E = 16384, 200, 64
_R = 128


def _body(ts_ref, emb_ref, out_ref):
    ts = ts_ref[...]
    ib = jax.lax.bitcast_convert_type(ts, jnp.int32)
    ikey = ib ^ ((ib >> 31) & jnp.int32(0x7FFFFFFF))
    ikey = jnp.where(ts == 0.0, 0, ikey)
    a = ikey[:, :, None]
    b = ikey[:, None, :]
    k2 = jax.lax.broadcasted_iota(jnp.int32, (_S, _S), 1)
    j2 = jax.lax.broadcasted_iota(jnp.int32, (_S, _S), 0)
    tri01 = (k2 < j2).astype(jnp.int32)[None, :, :]
    cmp = b < (a + tri01)
    rank = jnp.sum(cmp.astype(jnp.float32), axis=2)
    i2 = j2.astype(jnp.float32)[None, :, :]
    p = (i2 == rank[:, None, :]).astype(jnp.bfloat16)
    out = jnp.dot(p.reshape(_R * _S, _S), emb_ref[...].astype(jnp.bfloat16),
                  preferred_element_type=jnp.float32)
    out_ref[...] = out


def kernel(ts, pos_embeddings):
    out = pl.pallas_call(
        _body,
        grid=(_B // _R,),
        in_specs=[
            pl.BlockSpec((_R, _S), lambda i: (i, 0)),
            pl.BlockSpec((_S, _E), lambda i: (0, 0)),
        ],
        out_specs=pl.BlockSpec((_R * _S, _E), lambda i: (i, 0)),
        out_shape=jax.ShapeDtypeStruct((_B * _S, _E), jnp.float32),
    )(ts, pos_embeddings)
    return out.reshape(_B, _S, _E)
